# baseline (speedup 1.0000x reference)
import functools

import jax
import jax.numpy as jnp
from jax import lax
from jax.experimental import pallas as pl
from jax.experimental.pallas import tpu as pltpu
from jax.experimental.pallas import tpu_sc as plsc

VOCAB = 1000000
DIM = 64
BATCH = 4096
SEQ = 200

_INFO = plsc.get_sparse_core_info()
NC = _INFO.num_cores
NS = _INFO.num_subcores
NW = NC * NS

B = BATCH * SEQ
B_PER_W = B // NW
CHUNK = 800
NCHUNK = B_PER_W // CHUNK

_mesh = plsc.VectorSubcoreMesh(core_axis_name="c", subcore_axis_name="s")


@functools.partial(
    pl.kernel,
    mesh=_mesh,
    out_type=jax.ShapeDtypeStruct((B, DIM), jnp.float32),
    scratch_types=[
        pltpu.VMEM((CHUNK,), jnp.int32),
        pltpu.VMEM((CHUNK, DIM), jnp.float32),
        pltpu.SemaphoreType.DMA,
    ],
    compiler_params=pltpu.CompilerParams(use_tc_tiling_on_sc=False),
)
def _gather_kernel(idx_hbm, table_hbm, out_hbm, idx_v, rows_v, sem):
    wid = lax.axis_index("s") * NC + lax.axis_index("c")
    base = wid * B_PER_W

    def chunk_body(i, carry):
        off = pl.multiple_of(base + i * CHUNK, 8)
        pltpu.sync_copy(idx_hbm.at[pl.ds(off, CHUNK)], idx_v)
        pltpu.async_copy(table_hbm.at[idx_v], rows_v, sem).wait()
        pltpu.sync_copy(rows_v, out_hbm.at[pl.ds(off, CHUNK)])
        return carry

    lax.fori_loop(0, NCHUNK, chunk_body, 0)


def kernel(input, table):
    idx = input.reshape(B)
    out = _gather_kernel(idx, table)
    return out.reshape(BATCH, SEQ, DIM)


# speedup vs baseline: 1.0262x; 1.0262x over previous
import functools

import jax
import jax.numpy as jnp
from jax import lax
from jax.experimental import pallas as pl
from jax.experimental.pallas import tpu as pltpu
from jax.experimental.pallas import tpu_sc as plsc

VOCAB = 1000000
DIM = 64
BATCH = 4096
SEQ = 200

_INFO = plsc.get_sparse_core_info()
NC = _INFO.num_cores
NS = _INFO.num_subcores
NW = NC * NS

B = BATCH * SEQ
B_PER_W = B // NW
CHUNK = 400
NCHUNK = B_PER_W // CHUNK
NBUF = 4

_mesh = plsc.VectorSubcoreMesh(core_axis_name="c", subcore_axis_name="s")


@functools.partial(
    pl.kernel,
    mesh=_mesh,
    out_type=jax.ShapeDtypeStruct((B, DIM), jnp.float32),
    scratch_types=[
        pltpu.VMEM((B_PER_W,), jnp.int32),
        [pltpu.VMEM((CHUNK, DIM), jnp.float32) for _ in range(NBUF)],
        [pltpu.SemaphoreType.DMA for _ in range(NBUF)],
        [pltpu.SemaphoreType.DMA for _ in range(NBUF)],
    ],
    compiler_params=pltpu.CompilerParams(use_tc_tiling_on_sc=False),
)
def _gather_kernel(idx_hbm, table_hbm, out_hbm, idx_all, rows, gsem, ssem):
    wid = lax.axis_index("s") * NC + lax.axis_index("c")
    base = pl.multiple_of(wid * B_PER_W, 8)

    pltpu.sync_copy(idx_hbm.at[pl.ds(base, B_PER_W)], idx_all)

    def idx_slice(g):
        return idx_all.at[pl.ds(pl.multiple_of(g * CHUNK, 8), CHUNK)]

    for b in range(NBUF):
        pltpu.async_copy(table_hbm.at[idx_slice(b)], rows[b], gsem[b])

    @pl.loop(0, NCHUNK, step=NBUF)
    def _outer(g0):
        for b in range(NBUF):
            g = g0 + b
            pltpu.make_async_copy(
                table_hbm.at[idx_slice(g)], rows[b], gsem[b]
            ).wait()
            out_slice = out_hbm.at[pl.ds(base + pl.multiple_of(g * CHUNK, 8), CHUNK)]
            pltpu.async_copy(rows[b], out_slice, ssem[b]).wait()
            @pl.when(g + NBUF < NCHUNK)
            def _():
                pltpu.async_copy(
                    table_hbm.at[idx_slice(g + NBUF)], rows[b], gsem[b]
                )


def kernel(input, table):
    idx = input.reshape(B)
    out = _gather_kernel(idx, table)
    return out.reshape(BATCH, SEQ, DIM)
